# baseline (speedup 1.0000x reference)
import functools

import jax
import jax.numpy as jnp
from jax import lax
from jax.experimental import pallas as pl
from jax.experimental.pallas import tpu as pltpu
from jax.experimental.pallas import tpu_sc as plsc

_T = 1000
_D = 128
_B = 16384

_NC = 2
_NS = 16
_NW = _NC * _NS
_BPW = _B // _NW

_CH = 64
_NCH = _BPW // _CH


def _proj_silu_kernel(emb_ref, w_ref, b_ref, out_ref):
    x = lax.dot_general(
        emb_ref[...], w_ref[...],
        dimension_numbers=(((1,), (1,)), ((), ())),
        preferred_element_type=jnp.float32,
    ) + b_ref[...]
    out_ref[...] = x * jax.nn.sigmoid(x)


def _project_table(embedding, W1, b1):
    return pl.pallas_call(
        _proj_silu_kernel,
        out_shape=jax.ShapeDtypeStruct((_T, _D), jnp.float32),
    )(embedding, W1, b1.reshape(1, _D))


_vector_mesh = plsc.VectorSubcoreMesh(core_axis_name="c", subcore_axis_name="s")


@functools.partial(
    pl.kernel,
    mesh=_vector_mesh,
    out_type=jax.ShapeDtypeStruct((_B, _D), jnp.float32),
    scratch_types=[
        pltpu.VMEM((_NCH, _CH), jnp.int32),
        pltpu.VMEM((_NCH, _CH, _D), jnp.float32),
        pltpu.VMEM_SHARED((_T, _D), jnp.float32),
        pltpu.SemaphoreType.DMA((_NCH,)),
        pltpu.SemaphoreType.DMA((_NCH,)),
    ],
)
def _gather_kernel(table_hbm, idx_hbm, out_hbm, idx_v, bufs, table_sp,
                   gsem, wsem):
    cid = lax.axis_index("c")
    sid = lax.axis_index("s")
    wid = sid * _NC + cid
    base = wid * _BPW

    @pl.when(sid < _NS - 1)
    def _():
        pltpu.sync_copy(table_hbm.at[pl.ds(sid * 64, 64)],
                        table_sp.at[pl.ds(sid * 64, 64)])

    @pl.when(sid == _NS - 1)
    def _():
        pltpu.sync_copy(table_hbm.at[pl.ds(960, _T - 960)],
                        table_sp.at[pl.ds(960, _T - 960)])

    for i in range(_NCH):
        pltpu.sync_copy(idx_hbm.at[pl.ds(base + i * _CH, _CH)], idx_v.at[i])

    plsc.subcore_barrier()

    gathers = [
        pltpu.async_copy(table_sp.at[idx_v.at[i]], bufs.at[i], gsem.at[i])
        for i in range(_NCH)
    ]
    writes = []
    for i in range(_NCH):
        gathers[i].wait()
        writes.append(
            pltpu.async_copy(bufs.at[i],
                             out_hbm.at[pl.ds(base + i * _CH, _CH)],
                             wsem.at[i]))
    for wr in writes:
        wr.wait()


def kernel(diffusion_step, embedding, W1, b1):
    table = _project_table(embedding, W1, b1)
    idx = jnp.asarray(diffusion_step, jnp.int32)
    return _gather_kernel(table, idx)


# speedup vs baseline: 1.1035x; 1.1035x over previous
import functools

import jax
import jax.numpy as jnp
from jax import lax
from jax.experimental import pallas as pl
from jax.experimental.pallas import tpu as pltpu
from jax.experimental.pallas import tpu_sc as plsc

_T = 1000
_D = 128
_B = 16384

_NC = 2
_NS = 16
_NW = _NC * _NS
_BPW = _B // _NW

_CH = 128
_NCH = _BPW // _CH


def _proj_silu_kernel(emb_ref, w_ref, b_ref, out_ref):
    x = lax.dot_general(
        emb_ref[...], w_ref[...],
        dimension_numbers=(((1,), (1,)), ((), ())),
        preferred_element_type=jnp.float32,
    ) + b_ref[...]
    out_ref[...] = x * jax.nn.sigmoid(x)


def _project_table(embedding, W1, b1):
    return pl.pallas_call(
        _proj_silu_kernel,
        out_shape=jax.ShapeDtypeStruct((_T, _D), jnp.float32),
    )(embedding, W1, b1.reshape(1, _D))


_vector_mesh = plsc.VectorSubcoreMesh(core_axis_name="c", subcore_axis_name="s")


@functools.partial(
    pl.kernel,
    mesh=_vector_mesh,
    out_type=jax.ShapeDtypeStruct((_B, _D), jnp.float32),
    scratch_types=[
        pltpu.VMEM((_NCH, _CH), jnp.int32),
        pltpu.VMEM((_NCH, _CH, _D), jnp.float32),
        pltpu.VMEM_SHARED((_T, _D), jnp.float32),
        pltpu.SemaphoreType.DMA((_NCH,)),
        pltpu.SemaphoreType.DMA((_NCH,)),
        pltpu.SemaphoreType.DMA,
    ],
)
def _gather_kernel(table_hbm, idx_hbm, out_hbm, idx_v, bufs, table_sp,
                   gsem, wsem, tsem):
    cid = lax.axis_index("c")
    sid = lax.axis_index("s")
    wid = sid * _NC + cid
    base = wid * _BPW

    stage_base = jnp.minimum(sid * 64, _T - 64)
    stage = pltpu.async_copy(table_hbm.at[pl.ds(stage_base, 64)],
                             table_sp.at[pl.ds(stage_base, 64)], tsem)

    for i in range(_NCH):
        pltpu.sync_copy(idx_hbm.at[pl.ds(base + i * _CH, _CH)], idx_v.at[i])

    stage.wait()
    plsc.subcore_barrier()

    gathers = [
        pltpu.async_copy(table_sp.at[idx_v.at[i]], bufs.at[i], gsem.at[i])
        for i in range(_NCH)
    ]
    writes = []
    for i in range(_NCH):
        gathers[i].wait()
        writes.append(
            pltpu.async_copy(bufs.at[i],
                             out_hbm.at[pl.ds(base + i * _CH, _CH)],
                             wsem.at[i]))
    for wr in writes:
        wr.wait()


def kernel(diffusion_step, embedding, W1, b1):
    table = _project_table(embedding, W1, b1)
    idx = jnp.asarray(diffusion_step, jnp.int32)
    return _gather_kernel(table, idx)


# speedup vs baseline: 1.1381x; 1.0313x over previous
import functools

import jax
import jax.numpy as jnp
from jax import lax
from jax.experimental import pallas as pl
from jax.experimental.pallas import tpu as pltpu
from jax.experimental.pallas import tpu_sc as plsc

_T = 1000
_D = 128
_B = 16384

_NC = 2
_NS = 16
_NW = _NC * _NS
_BPW = _B // _NW

_CH = 128
_NCH = _BPW // _CH


def _proj_silu_kernel(emb_ref, w_ref, b_ref, out_ref):
    x = lax.dot_general(
        emb_ref[...], w_ref[...],
        dimension_numbers=(((1,), (1,)), ((), ())),
        preferred_element_type=jnp.float32,
    ) + b_ref[...]
    out_ref[...] = x * jax.nn.sigmoid(x)


def _project_table(embedding, W1, b1):
    return pl.pallas_call(
        _proj_silu_kernel,
        out_shape=jax.ShapeDtypeStruct((_T, _D), jnp.float32),
    )(embedding, W1, b1.reshape(1, _D))


_vector_mesh = plsc.VectorSubcoreMesh(core_axis_name="c", subcore_axis_name="s")


@functools.partial(
    pl.kernel,
    mesh=_vector_mesh,
    out_type=jax.ShapeDtypeStruct((_B, _D), jnp.float32),
    scratch_types=[
        pltpu.VMEM((_NCH, _CH), jnp.int32),
        pltpu.VMEM((_NCH, _CH, _D), jnp.float32),
        pltpu.VMEM_SHARED((_T, _D), jnp.float32),
        pltpu.SemaphoreType.DMA((_NCH,)),
        pltpu.SemaphoreType.DMA((_NCH,)),
        pltpu.SemaphoreType.DMA,
    ],
)
def _gather_kernel(table_hbm, idx_hbm, out_hbm, idx_v, bufs, table_sp,
                   gsem, wsem, tsem):
    cid = lax.axis_index("c")
    sid = lax.axis_index("s")
    wid = sid * _NC + cid
    base = wid * _BPW

    stage_base = jnp.minimum(sid * 64, _T - 64)
    stage = pltpu.async_copy(table_hbm.at[pl.ds(stage_base, 64)],
                             table_sp.at[pl.ds(stage_base, 64)], tsem)

    idx_copies = [
        pltpu.async_copy(idx_hbm.at[pl.ds(base + i * _CH, _CH)], idx_v.at[i],
                         wsem.at[i])
        for i in range(_NCH)
    ]

    stage.wait()
    plsc.subcore_barrier()

    gathers = []
    for i in range(_NCH):
        idx_copies[i].wait()
        gathers.append(
            pltpu.async_copy(table_sp.at[idx_v.at[i]], bufs.at[i], gsem.at[i]))
    writes = []
    for i in range(_NCH):
        gathers[i].wait()
        writes.append(
            pltpu.async_copy(bufs.at[i],
                             out_hbm.at[pl.ds(base + i * _CH, _CH)],
                             wsem.at[i]))
    for wr in writes:
        wr.wait()


def kernel(diffusion_step, embedding, W1, b1):
    table = _project_table(embedding, W1, b1)
    idx = jnp.asarray(diffusion_step, jnp.int32)
    return _gather_kernel(table, idx)
